# Initial kernel scaffold; baseline (speedup 1.0000x reference)
import jax
import jax.numpy as jnp
from jax.experimental import pallas as pl


def kernel(word_encoding, char_encoding, word_table, char_table):
    raise NotImplementedError("write your pallas kernel here")


# speedup vs baseline: 1.4394x; 1.4394x over previous
import functools

import jax
import jax.numpy as jnp
from jax import lax
from jax.experimental import pallas as pl
from jax.experimental.pallas import tpu as pltpu, tpu_sc as plsc

NC = 2
NS = 16
L = 16
NW = NC * NS

B = 1024
W = 50
C = 16
DW = 300
DWP = 384
REC = 128
NR = DWP // REC
DC = 64
DOUT = DW + DC
TOK = B * W
TPW = TOK // NW
CHUNK = 80
NCHUNK = TPW // CHUNK
G = CHUNK // L
RLEN = (REC, REC, DW - 2 * REC)


def _body(widx_hbm, cidx_hbm, wtab_hbm, ctab_hbm, out_hbm,
          widx_v, cidx_v, rec_v, rows_v, comb_v, ctab_v, sem_g):
    wid = lax.axis_index("s") * NC + lax.axis_index("c")
    base0 = wid * TPW

    pltpu.sync_copy(ctab_hbm, ctab_v)
    zeros = jnp.zeros((L,), jnp.float32)
    for j in range(DC // L):
        ctab_v[0, pl.ds(j * L, L)] = zeros

    pltpu.sync_copy(widx_hbm.at[pl.ds(base0, TPW)], widx_v)
    pltpu.sync_copy(cidx_hbm.at[:, pl.ds(base0, TPW)], cidx_v)

    def rgroup(i, carry):
        off = pl.multiple_of(i * L, 8)
        w = widx_v[pl.ds(off, L)]
        w3 = w + w + w
        rec_v[0, pl.ds(off, L)] = w3
        rec_v[1, pl.ds(off, L)] = w3 + 1
        rec_v[2, pl.ds(off, L)] = w3 + 2
        return carry

    lax.fori_loop(0, TPW // L, rgroup, 0)

    lane_iota = lax.iota(jnp.int32, L)

    def chunk_body(k, carry0):
        lbase = pl.multiple_of(k * CHUNK, 8)
        base = base0 + lbase
        gathers = [
            pltpu.async_copy(
                wtab_hbm.at[rec_v.at[t, pl.ds(lbase, CHUNK)]],
                rows_v.at[pl.ds(t * CHUNK, CHUNK), :], sem_g)
            for t in range(NR)
        ]

        def pool_group(g, carry):
            gbase = pl.multiple_of(lbase + g * L, 8)
            cvecs = [cidx_v[c, pl.ds(gbase, L)] for c in range(C)]
            tok = lane_iota + g * L

            def dbody(d, c2):
                dcol = jnp.full((L,), d, jnp.int32)
                acc = plsc.load_gather(ctab_v, [cvecs[0], dcol])
                for c in range(1, C):
                    acc = jnp.maximum(
                        acc, plsc.load_gather(ctab_v, [cvecs[c], dcol]))
                plsc.store_scatter(comb_v, [tok, dcol + DW], acc)
                return c2

            return lax.fori_loop(0, DC, dbody, carry, unroll=2)

        lax.fori_loop(0, G, pool_group, 0)

        for gth in gathers:
            gth.wait()

        def copy_group(g, carry):
            tok = lane_iota + g * L
            for t in range(NR):
                tokr = tok + t * CHUNK

                def wbody(d, c2, _t=t, _tokr=tokr):
                    dcol = jnp.full((L,), d, jnp.int32)
                    vals = plsc.load_gather(rows_v, [_tokr, dcol])
                    plsc.store_scatter(
                        comb_v, [tok, dcol + _t * REC], vals)
                    return c2

                carry = lax.fori_loop(0, RLEN[t], wbody, carry, unroll=4)
            return carry

        lax.fori_loop(0, G, copy_group, 0)

        pltpu.sync_copy(comb_v, out_hbm.at[pl.ds(base, CHUNK), :])
        return carry0

    lax.fori_loop(0, NCHUNK, chunk_body, 0)


@functools.partial(jax.jit, static_argnames=())
def kernel(word_encoding, char_encoding, word_table, char_table):
    widx = word_encoding.reshape(TOK).astype(jnp.int32)
    cidx = char_encoding.reshape(TOK, C).T.astype(jnp.int32)
    wtabr = jnp.pad(word_table, ((0, 0), (0, DWP - DW))).reshape(-1, REC)

    mesh = plsc.VectorSubcoreMesh(core_axis_name="c", subcore_axis_name="s")
    run = pl.kernel(
        _body,
        out_type=jax.ShapeDtypeStruct((TOK, DOUT), jnp.float32),
        mesh=mesh,
        compiler_params=pltpu.CompilerParams(
            use_tc_tiling_on_sc=False, needs_layout_passes=False),
        scratch_types=[
            pltpu.VMEM((TPW,), jnp.int32),
            pltpu.VMEM((C, TPW), jnp.int32),
            pltpu.VMEM((NR, TPW), jnp.int32),
            pltpu.VMEM((NR * CHUNK, REC), jnp.float32),
            pltpu.VMEM((CHUNK, DOUT), jnp.float32),
            pltpu.VMEM((128, DC), jnp.float32),
            pltpu.SemaphoreType.DMA,
        ],
    )
    out = run(widx, cidx, wtabr, char_table)
    return out.reshape(B, W, DOUT)
